# Initial kernel scaffold; baseline (speedup 1.0000x reference)
import jax
import jax.numpy as jnp
from jax.experimental import pallas as pl


def kernel(x):
    raise NotImplementedError("write your pallas kernel here")


# speedup vs baseline: 10.0413x; 10.0413x over previous
import functools

import numpy as np
import jax
import jax.numpy as jnp
from jax import lax
from jax.experimental import pallas as pl
from jax.experimental.pallas import tpu as pltpu
from jax.experimental.pallas import tpu_sc as plsc

_B, _T, _F = 16, 4096, 256
_N = _B * _T
_P = 0.1
_SEED = 0


def _threefry_pair(kp, x0, x1):
    rotations = [(13, 15, 26, 6), (17, 29, 16, 24)]

    def rotl(v, d):
        return (v << np.uint32(d)) | (v >> np.uint32(32 - d))

    x0 = x0.copy()
    x1 = x1.copy()
    k0, k1 = np.uint32(kp[0]), np.uint32(kp[1])
    k2 = k0 ^ k1 ^ np.uint32(0x1BD11BDA)
    ks = [k0, k1, k2]
    x0 += ks[0]
    x1 += ks[1]
    for r in range(5):
        for d in rotations[r % 2]:
            x0 += x1
            x1 = rotl(x1, d)
            x1 ^= x0
        x0 += ks[(r + 1) % 3]
        x1 += ks[(r + 2) % 3]
        x1 += np.uint32(r + 1)
    return x0, x1


def _uniform_bits(kp, n):
    lo = np.arange(n, dtype=np.uint32)
    hi = np.zeros(n, dtype=np.uint32)
    x0, x1 = _threefry_pair(kp, hi, lo)
    bits = x0 ^ x1
    return ((bits >> np.uint32(9)) | np.uint32(0x3F800000)).view(np.float32) - np.float32(1.0)


def _split2(kp):
    x0, x1 = _threefry_pair(kp, np.zeros(2, np.uint32), np.arange(2, dtype=np.uint32))
    return (x0[0], x1[0]), (x0[1], x1[1])


def _build_perm():
    key = (np.uint32(_SEED >> 32), np.uint32(_SEED & 0xFFFFFFFF))
    k1, k2 = _split2(key)
    n = _N * _F
    swap = _uniform_bits(k1, n).reshape(_N, _F) < np.float32(_P)
    rk = _uniform_bits(k2, n).reshape(_N, _F)
    base = np.arange(_F, dtype=np.float32)
    scores = np.where(swap, rk, base[None, :])
    return np.argsort(scores, axis=-1, kind="stable").astype(np.int32)


def _build_packed():
    perm = _build_perm()
    p = perm.reshape(_N, 4, 4, 16).astype(np.uint32)
    pw = (p[:, :, 0] | (p[:, :, 1] << 8) | (p[:, :, 2] << 16) | (p[:, :, 3] << 24))
    return pw.reshape(_N, 64).astype(np.int32)


_PACKED = _build_packed()


_NC, _NS, _L = 2, 16, 16
_NW = _NC * _NS
_RPW = _N // _NW
_R = 128
_CHUNKS = _RPW // _R


def _sc_body(x_hbm, pw_hbm, out_hbm, x_v, pw_v, out_v):
    wid = lax.axis_index("c") * _NS + lax.axis_index("s")

    def chunk_body(c, carry):
        base = wid * _RPW + c * _R
        pltpu.sync_copy(x_hbm.at[pl.ds(base * _F, _R * _F)], x_v)
        pltpu.sync_copy(pw_hbm.at[pl.ds(base * 64, _R * 64)], pw_v)

        def row_body(r, carry2):
            rbase = r * _F
            for i in range(4):
                pk = pw_v[pl.ds(r * 64 + 16 * i, 16)]
                for k in range(4):
                    idx = ((pk >> (8 * k)) & 255) + rbase
                    vals = plsc.load_gather(x_v, [idx])
                    out_v[pl.ds(rbase + 64 * i + 16 * k, 16)] = vals
            return carry2

        lax.fori_loop(0, _R, row_body, 0)
        pltpu.sync_copy(out_v, out_hbm.at[pl.ds(base * _F, _R * _F)])
        return carry

    lax.fori_loop(0, _CHUNKS, chunk_body, 0)


def kernel(x):
    x1 = x.reshape(_N * _F)
    pw = jnp.asarray(_PACKED.reshape(_N * 64))
    mesh = plsc.VectorSubcoreMesh(core_axis_name="c", subcore_axis_name="s")
    f = pl.kernel(
        _sc_body,
        mesh=mesh,
        out_type=jax.ShapeDtypeStruct((_N * _F,), jnp.float32),
        scratch_types=[
            pltpu.VMEM((_R * _F,), jnp.float32),
            pltpu.VMEM((_R * 64,), jnp.int32),
            pltpu.VMEM((_R * _F,), jnp.float32),
        ],
        compiler_params=pltpu.CompilerParams(needs_layout_passes=False),
    )
    out = f(x1, pw)
    return out.reshape(_B, _T, _F)


# speedup vs baseline: 18.7644x; 1.8687x over previous
import functools

import numpy as np
import jax
import jax.numpy as jnp
from jax import lax
from jax.experimental import pallas as pl
from jax.experimental.pallas import tpu as pltpu
from jax.experimental.pallas import tpu_sc as plsc

_B, _T, _F = 16, 4096, 256
_N = _B * _T
_P = 0.1
_SEED = 0


def _threefry_pair(kp, x0, x1):
    rotations = [(13, 15, 26, 6), (17, 29, 16, 24)]

    def rotl(v, d):
        return (v << np.uint32(d)) | (v >> np.uint32(32 - d))

    x0 = x0.copy()
    x1 = x1.copy()
    k0, k1 = np.uint32(kp[0]), np.uint32(kp[1])
    k2 = k0 ^ k1 ^ np.uint32(0x1BD11BDA)
    ks = [k0, k1, k2]
    x0 += ks[0]
    x1 += ks[1]
    for r in range(5):
        for d in rotations[r % 2]:
            x0 += x1
            x1 = rotl(x1, d)
            x1 ^= x0
        x0 += ks[(r + 1) % 3]
        x1 += ks[(r + 2) % 3]
        x1 += np.uint32(r + 1)
    return x0, x1


def _uniform_bits(kp, n):
    lo = np.arange(n, dtype=np.uint32)
    hi = np.zeros(n, dtype=np.uint32)
    x0, x1 = _threefry_pair(kp, hi, lo)
    bits = x0 ^ x1
    return ((bits >> np.uint32(9)) | np.uint32(0x3F800000)).view(np.float32) - np.float32(1.0)


def _split2(kp):
    x0, x1 = _threefry_pair(kp, np.zeros(2, np.uint32), np.arange(2, dtype=np.uint32))
    return (x0[0], x1[0]), (x0[1], x1[1])


def _build_perm():
    key = (np.uint32(_SEED >> 32), np.uint32(_SEED & 0xFFFFFFFF))
    k1, k2 = _split2(key)
    n = _N * _F
    swap = _uniform_bits(k1, n).reshape(_N, _F) < np.float32(_P)
    rk = _uniform_bits(k2, n).reshape(_N, _F)
    base = np.arange(_F, dtype=np.float32)
    scores = np.where(swap, rk, base[None, :])
    return np.argsort(scores, axis=-1, kind="stable").astype(np.int32)


def _build_packed():
    perm = _build_perm()
    p = perm.reshape(_N, 4, 4, 16).astype(np.uint32)
    pw = (p[:, :, 0] | (p[:, :, 1] << 8) | (p[:, :, 2] << 16) | (p[:, :, 3] << 24))
    return pw.reshape(_N, 64).astype(np.int32)


_PACKED = _build_packed()


_NC, _NS, _L = 2, 16, 16
_NW = _NC * _NS
_RPW = _N // _NW
_R = 64
_CHUNKS = _RPW // _R
_ILV = 2


def _sc_body(x_hbm, pw_hbm, out_hbm,
             x_v0, x_v1, pw_v0, pw_v1, o_v0, o_v1,
             sx0, sx1, sp0, sp1, so0, so1):
    wid = lax.axis_index("c") * _NS + lax.axis_index("s")
    row0 = wid * _RPW

    x_v = (x_v0, x_v1)
    pw_v = (pw_v0, pw_v1)
    o_v = (o_v0, o_v1)
    sx = (sx0, sx1)
    sp = (sp0, sp1)
    so = (so0, so1)

    def in_desc(c, s):
        base = row0 + c * _R
        cx = pltpu.make_async_copy(x_hbm.at[pl.ds(base * _F, _R * _F)], x_v[s], sx[s])
        cp = pltpu.make_async_copy(pw_hbm.at[pl.ds(base * 64, _R * 64)], pw_v[s], sp[s])
        return cx, cp

    def out_desc(c, s):
        base = row0 + c * _R
        return pltpu.make_async_copy(o_v[s], out_hbm.at[pl.ds(base * _F, _R * _F)], so[s])

    def start_in(c, s):
        cx, cp = in_desc(c, s)
        cx.start()
        cp.start()

    def wait_in(c, s):
        cx, cp = in_desc(c, s)
        cx.wait()
        cp.wait()

    def compute(s):
        xb, pb, ob = x_v[s], pw_v[s], o_v[s]

        def row_body(r2, carry):
            r = r2 * _ILV
            work = []
            for rr in range(_ILV):
                rbase = (r + rr) * _F
                for i in range(4):
                    pk = pb[pl.ds((r + rr) * 64 + 16 * i, 16)]
                    for k in range(4):
                        idx = ((pk >> (8 * k)) & 255) + rbase
                        work.append((rbase + 64 * i + 16 * k, idx))
            vals = [plsc.load_gather(xb, [idx]) for _, idx in work]
            for (off, _), v in zip(work, vals):
                ob[pl.ds(off, 16)] = v
            return carry

        lax.fori_loop(0, _R // _ILV, row_body, 0)

    start_in(0, 0)
    start_in(1, 1)
    wait_in(0, 0)
    compute(0)
    out_desc(0, 0).start()
    start_in(2, 0)
    wait_in(1, 1)
    compute(1)
    out_desc(1, 1).start()
    start_in(3, 1)

    def pair_body(c2, carry):
        for b in range(2):
            c = 2 * c2 + b
            wait_in(c, b)
            out_desc(c - 2, b).wait()
            compute(b)
            out_desc(c, b).start()
            start_in(c + 2, b)
        return carry

    lax.fori_loop(1, _CHUNKS // 2 - 1, pair_body, 0)

    for b in range(2):
        c = _CHUNKS - 2 + b
        wait_in(c, b)
        out_desc(c - 2, b).wait()
        compute(b)
        out_desc(c, b).start()
    out_desc(_CHUNKS - 2, 0).wait()
    out_desc(_CHUNKS - 1, 1).wait()


def kernel(x):
    x1 = x.reshape(_N * _F)
    pw = jnp.asarray(_PACKED.reshape(_N * 64))
    mesh = plsc.VectorSubcoreMesh(core_axis_name="c", subcore_axis_name="s")
    f = pl.kernel(
        _sc_body,
        mesh=mesh,
        out_type=jax.ShapeDtypeStruct((_N * _F,), jnp.float32),
        scratch_types=[
            pltpu.VMEM((_R * _F,), jnp.float32),
            pltpu.VMEM((_R * _F,), jnp.float32),
            pltpu.VMEM((_R * 64,), jnp.int32),
            pltpu.VMEM((_R * 64,), jnp.int32),
            pltpu.VMEM((_R * _F,), jnp.float32),
            pltpu.VMEM((_R * _F,), jnp.float32),
            pltpu.SemaphoreType.DMA,
            pltpu.SemaphoreType.DMA,
            pltpu.SemaphoreType.DMA,
            pltpu.SemaphoreType.DMA,
            pltpu.SemaphoreType.DMA,
            pltpu.SemaphoreType.DMA,
        ],
        compiler_params=pltpu.CompilerParams(needs_layout_passes=False),
    )
    out = f(x1, pw)
    return out.reshape(_B, _T, _F)


# speedup vs baseline: 39.0932x; 2.0834x over previous
import functools

import numpy as np
import jax
import jax.numpy as jnp
from jax import lax
from jax.experimental import pallas as pl
from jax.experimental.pallas import tpu as pltpu
from jax.experimental.pallas import tpu_sc as plsc

_B, _T, _F = 16, 4096, 256
_N = _B * _T
_P = 0.1
_SEED = 0


def _threefry_pair(kp, x0, x1):
    rotations = [(13, 15, 26, 6), (17, 29, 16, 24)]

    def rotl(v, d):
        return (v << np.uint32(d)) | (v >> np.uint32(32 - d))

    x0 = x0.copy()
    x1 = x1.copy()
    k0, k1 = np.uint32(kp[0]), np.uint32(kp[1])
    k2 = k0 ^ k1 ^ np.uint32(0x1BD11BDA)
    ks = [k0, k1, k2]
    x0 += ks[0]
    x1 += ks[1]
    for r in range(5):
        for d in rotations[r % 2]:
            x0 += x1
            x1 = rotl(x1, d)
            x1 ^= x0
        x0 += ks[(r + 1) % 3]
        x1 += ks[(r + 2) % 3]
        x1 += np.uint32(r + 1)
    return x0, x1


def _uniform_bits(kp, n):
    lo = np.arange(n, dtype=np.uint32)
    hi = np.zeros(n, dtype=np.uint32)
    x0, x1 = _threefry_pair(kp, hi, lo)
    bits = x0 ^ x1
    return ((bits >> np.uint32(9)) | np.uint32(0x3F800000)).view(np.float32) - np.float32(1.0)


def _split2(kp):
    x0, x1 = _threefry_pair(kp, np.zeros(2, np.uint32), np.arange(2, dtype=np.uint32))
    return (x0[0], x1[0]), (x0[1], x1[1])


def _build_perm():
    key = (np.uint32(_SEED >> 32), np.uint32(_SEED & 0xFFFFFFFF))
    k1, k2 = _split2(key)
    n = _N * _F
    swap = _uniform_bits(k1, n).reshape(_N, _F) < np.float32(_P)
    rk = _uniform_bits(k2, n).reshape(_N, _F)
    base = np.arange(_F, dtype=np.float32)
    scores = np.where(swap, rk, base[None, :])
    return np.argsort(scores, axis=-1, kind="stable").astype(np.int32)


def _build_packed():
    perm = _build_perm().astype(np.uint32)
    v = ((perm >> 7) << 10) | (perm & 127)
    g = v.reshape(_N, 16, 16)
    pw = g[:, 0::2, :] | (g[:, 1::2, :] << 16)
    return pw.reshape(_N, 128).astype(np.int32)


_PACKED = _build_packed()


_NC, _NS, _L = 2, 16, 16
_NW = _NC * _NS
_RPW = _N // _NW
_R = 64
_CHUNKS = _RPW // _R
_ILV = 2


def _sc_body(x_hbm, pw_hbm, out_hbm,
             x_v0, x_v1, pw_v0, pw_v1, o_v0, o_v1,
             sx0, sx1, sp0, sp1, so0, so1):
    wid = lax.axis_index("c") * _NS + lax.axis_index("s")
    row0 = wid * _RPW

    x_v = (x_v0, x_v1)
    pw_v = (pw_v0, pw_v1)
    o_v = (o_v0, o_v1)
    sx = (sx0, sx1)
    sp = (sp0, sp1)
    so = (so0, so1)

    def in_desc(c, s):
        base = row0 + c * _R
        cx = pltpu.make_async_copy(x_hbm.at[pl.ds(base * _F, _R * _F)], x_v[s], sx[s])
        cp = pltpu.make_async_copy(pw_hbm.at[pl.ds(base * 128, _R * 128)], pw_v[s], sp[s])
        return cx, cp

    def out_desc(c, s):
        base = row0 + c * _R
        return pltpu.make_async_copy(o_v[s], out_hbm.at[pl.ds(base * _F, _R * _F)], so[s])

    def start_in(c, s):
        cx, cp = in_desc(c, s)
        cx.start()
        cp.start()

    def wait_in(c, s):
        cx, cp = in_desc(c, s)
        cx.wait()
        cp.wait()

    def compute(s):
        xb, pb, ob = x_v[s], pw_v[s], o_v[s]

        def row_body(r2, carry):
            r = r2 * _ILV
            work = []
            for rr in range(_ILV):
                rl = r + rr
                rbase = (rl >> 3) * 2048 + (rl & 7) * 128
                for w in range(8):
                    pk = pb[pl.ds(rl * 128 + 16 * w, 16)]
                    for h in range(2):
                        g = 2 * w + h
                        v = (pk & 0xFFFF) if h == 0 else (pk >> 16)
                        idx = v + rbase
                        doff = (g // 8) * 1024 + 16 * (g % 8)
                        work.append((rbase + doff, idx))
            vals = [plsc.load_gather(xb, [idx]) for _, idx in work]
            for (off, _), v in zip(work, vals):
                ob[pl.ds(off, 16)] = v
            return carry

        lax.fori_loop(0, _R // _ILV, row_body, 0)

    start_in(0, 0)
    start_in(1, 1)
    wait_in(0, 0)
    compute(0)
    out_desc(0, 0).start()
    start_in(2, 0)
    wait_in(1, 1)
    compute(1)
    out_desc(1, 1).start()
    start_in(3, 1)

    def pair_body(c2, carry):
        for b in range(2):
            c = 2 * c2 + b
            wait_in(c, b)
            out_desc(c - 2, b).wait()
            compute(b)
            out_desc(c, b).start()
            start_in(c + 2, b)
        return carry

    lax.fori_loop(1, _CHUNKS // 2 - 1, pair_body, 0)

    for b in range(2):
        c = _CHUNKS - 2 + b
        wait_in(c, b)
        out_desc(c - 2, b).wait()
        compute(b)
        out_desc(c, b).start()
    out_desc(_CHUNKS - 2, 0).wait()
    out_desc(_CHUNKS - 1, 1).wait()


def kernel(x):
    x1 = x.reshape(_N // 8, 8, 2, 128).transpose(0, 2, 1, 3).reshape(_N * _F)
    pw = jnp.asarray(_PACKED.reshape(_N * 128))
    mesh = plsc.VectorSubcoreMesh(core_axis_name="c", subcore_axis_name="s")
    f = pl.kernel(
        _sc_body,
        mesh=mesh,
        out_type=jax.ShapeDtypeStruct((_N * _F,), jnp.float32),
        scratch_types=[
            pltpu.VMEM((_R * _F,), jnp.float32),
            pltpu.VMEM((_R * _F,), jnp.float32),
            pltpu.VMEM((_R * 128,), jnp.int32),
            pltpu.VMEM((_R * 128,), jnp.int32),
            pltpu.VMEM((_R * _F,), jnp.float32),
            pltpu.VMEM((_R * _F,), jnp.float32),
            pltpu.SemaphoreType.DMA,
            pltpu.SemaphoreType.DMA,
            pltpu.SemaphoreType.DMA,
            pltpu.SemaphoreType.DMA,
            pltpu.SemaphoreType.DMA,
            pltpu.SemaphoreType.DMA,
        ],
        compiler_params=pltpu.CompilerParams(needs_layout_passes=False),
    )
    out = f(x1, pw)
    return (out.reshape(_N // 8, 2, 8, 128)
               .transpose(0, 2, 1, 3)
               .reshape(_B, _T, _F))
